# baseline (speedup 1.0000x reference)
import functools

import jax
import jax.numpy as jnp
from jax import lax
from jax.experimental import pallas as pl
from jax.experimental.pallas import tpu as pltpu
from jax.experimental.pallas import tpu_sc as plsc

_D = 1024
_B = 16
_L = 512
_RW = 16
_CR = 2
_NCH = _RW // _CR


def _sc_body(src_t, tgt_t, o0, o1, o2, o3,
             bufA0, bufA1, zbuf, zidx, ix00, ix01, ix10, ix11,
             gsem0, gsem1, ssem0, ssem1):
    cid = lax.axis_index("c")
    sid = lax.axis_index("s")
    wid = sid * 2 + cid
    rb = jnp.minimum(wid * _RW, 511 - _RW)

    jobs = (
        (src_t, 0, o0), (src_t, 1, o1), (src_t, 2, o2),
        (tgt_t, 0, o3),
    )
    bufs = (bufA0, bufA1)
    gsems = (gsem0, gsem1)
    ssems = (ssem0, ssem1)
    idxs = ((ix00, ix01), (ix10, ix11))
    iota16 = lax.iota(jnp.int32, 16)
    zeros16 = jnp.zeros((16,), jnp.float32)

    @pl.when(wid == 0)
    def _():
        for r in range(16):
            for t in range(_D // 16):
                zbuf[r, pl.ds(t * 16, 16)] = zeros16
        zidx[pl.ds(0, 16)] = iota16 * _L
        for _, _l, out2d in jobs:
            pltpu.async_copy(zbuf, out2d.at[zidx], ssems[0]).wait()

    chunks = []
    for e_t, l, out2d in jobs:
        for g in range(_NCH):
            chunks.append((e_t, l, out2d, g))
    n = len(chunks)

    def gather(i):
        e_t, l, _, g = chunks[i]
        p = i % 2
        r0 = rb + g * _CR
        return pltpu.async_copy(
            e_t.at[pl.ds(r0, _CR), :, pl.ds(l * _D, _D)], bufs[p], gsems[p])

    def scatter(i):
        _, _, out2d, g = chunks[i]
        p = i % 2
        r0 = rb + g * _CR
        hs = []
        for r in range(_CR):
            ix = idxs[p][r]
            ix[pl.ds(0, 16)] = iota16 * _L + (r0 + r + 1)
            hs.append(pltpu.async_copy(bufs[p].at[r], out2d.at[ix], ssems[p]))
        return hs

    g = [None, None]
    s = [None, None]
    g[0] = gather(0)
    for i in range(n):
        p = i % 2
        q = (i + 1) % 2
        if i + 1 < n:
            if s[q] is not None:
                for h in s[q]:
                    h.wait()
                s[q] = None
            g[q] = gather(i + 1)
        g[p].wait()
        s[p] = scatter(i)
    for s_ in s:
        if s_ is not None:
            for h in s_:
                h.wait()


def _sc_part(src_t, tgt_t):
    mesh = plsc.VectorSubcoreMesh(core_axis_name="c", subcore_axis_name="s")
    out_struct = jax.ShapeDtypeStruct((_B * _L, _D), jnp.float32)
    kern = functools.partial(
        pl.kernel,
        out_type=[out_struct] * 4,
        mesh=mesh,
        scratch_types=[
            pltpu.VMEM((_CR, _B, _D), jnp.float32),
            pltpu.VMEM((_CR, _B, _D), jnp.float32),
            pltpu.VMEM((16, _D), jnp.float32),
            pltpu.VMEM((16,), jnp.int32),
            pltpu.VMEM((16,), jnp.int32),
            pltpu.VMEM((16,), jnp.int32),
            pltpu.VMEM((16,), jnp.int32),
            pltpu.VMEM((16,), jnp.int32),
            pltpu.SemaphoreType.DMA,
            pltpu.SemaphoreType.DMA,
            pltpu.SemaphoreType.DMA,
            pltpu.SemaphoreType.DMA,
        ],
    )(_sc_body)
    return kern(src_t, tgt_t)


def _tc_body(in_ref, o):
    o[0, 0:1, :] = jnp.zeros((1, _D), jnp.float32)
    o[0, 1:, :] = in_ref[0, :, :]


def _tc_layer(e, l):
    return pl.pallas_call(
        _tc_body,
        grid=(_B,),
        in_specs=[pl.BlockSpec((1, 511, _D), lambda b, _l=l: (b, 0, _l))],
        out_specs=pl.BlockSpec((1, _L, _D), lambda b: (b, 0, 0)),
        out_shape=jax.ShapeDtypeStruct((_B, _L, _D), jnp.float32),
    )(e)


def kernel(elmo_src, elmo_tgt):
    src_t = jnp.transpose(elmo_src, (1, 0, 2))
    tgt_t = jnp.transpose(elmo_tgt, (1, 0, 2))
    sc_outs = _sc_part(src_t, tgt_t)
    s0, s1, s2, t0 = (o.reshape(_B, _L, _D) for o in sc_outs)
    ts = elmo_tgt[:, :, _D:]
    t1 = _tc_layer(ts, 0)
    t2 = _tc_layer(ts, 1)
    return (s0, s1, s2, t0, t1, t2)


# speedup vs baseline: 1.4237x; 1.4237x over previous
import functools

import jax
import jax.numpy as jnp
from jax import lax
from jax.experimental import pallas as pl
from jax.experimental.pallas import tpu as pltpu
from jax.experimental.pallas import tpu_sc as plsc

_D = 1024
_B = 16
_L = 512
_RW = 16
_CR = 2
_NCH = _RW // _CR


def _sc_body(src_t, tgt_t, o0, o1, o2, o3, o4, o5,
             bufA0, bufA1, bufA2, zbuf, zidx,
             ix00, ix01, ix10, ix11, ix20, ix21,
             gsem0, gsem1, gsem2, ssem0, ssem1, ssem2):
    cid = lax.axis_index("c")
    sid = lax.axis_index("s")
    wid = sid * 2 + cid
    rb = jnp.minimum(wid * _RW, 511 - _RW)

    jobs = (
        (src_t, 0, o0), (src_t, 1, o1), (src_t, 2, o2),
        (tgt_t, 0, o3), (tgt_t, 1, o4), (tgt_t, 2, o5),
    )
    bufs = (bufA0, bufA1, bufA2)
    gsems = (gsem0, gsem1, gsem2)
    ssems = (ssem0, ssem1, ssem2)
    idxs = ((ix00, ix01), (ix10, ix11), (ix20, ix21))
    iota16 = lax.iota(jnp.int32, 16)
    zeros16 = jnp.zeros((16,), jnp.float32)

    @pl.when(wid == 0)
    def _():
        for r in range(16):
            for t in range(_D // 16):
                zbuf[r, pl.ds(t * 16, 16)] = zeros16
        zidx[pl.ds(0, 16)] = iota16 * _L
        for _, _l, out2d in jobs:
            pltpu.async_copy(zbuf, out2d.at[zidx], ssems[0]).wait()

    chunks = []
    for e_t, l, out2d in jobs:
        for g in range(_NCH):
            chunks.append((e_t, l, out2d, g))
    n = len(chunks)

    def gather(i):
        e_t, l, _, g = chunks[i]
        p = i % 3
        r0 = rb + g * _CR
        return pltpu.async_copy(
            e_t.at[pl.ds(r0, _CR), :, pl.ds(l * _D, _D)], bufs[p], gsems[p])

    def scatter(i):
        _, _, out2d, g = chunks[i]
        p = i % 3
        r0 = rb + g * _CR
        hs = []
        for r in range(_CR):
            ix = idxs[p][r]
            ix[pl.ds(0, 16)] = iota16 * _L + (r0 + r + 1)
            hs.append(pltpu.async_copy(bufs[p].at[r], out2d.at[ix], ssems[p]))
        return hs

    g = [None, None, None]
    s = [None, None, None]
    g[0] = gather(0)
    g[1] = gather(1)
    for i in range(n):
        p = i % 3
        if i + 2 < n:
            q = (i + 2) % 3
            if s[q] is not None:
                for h in s[q]:
                    h.wait()
                s[q] = None
            g[q] = gather(i + 2)
        g[p].wait()
        s[p] = scatter(i)
    for s_ in s:
        if s_ is not None:
            for h in s_:
                h.wait()


def kernel(elmo_src, elmo_tgt):
    mesh = plsc.VectorSubcoreMesh(core_axis_name="c", subcore_axis_name="s")
    out_struct = jax.ShapeDtypeStruct((_B * _L, _D), jnp.float32)
    kern = functools.partial(
        pl.kernel,
        out_type=[out_struct] * 6,
        mesh=mesh,
        scratch_types=[
            pltpu.VMEM((_CR, _B, _D), jnp.float32),
            pltpu.VMEM((_CR, _B, _D), jnp.float32),
            pltpu.VMEM((_CR, _B, _D), jnp.float32),
            pltpu.VMEM((16, _D), jnp.float32),
            pltpu.VMEM((16,), jnp.int32),
            pltpu.VMEM((16,), jnp.int32),
            pltpu.VMEM((16,), jnp.int32),
            pltpu.VMEM((16,), jnp.int32),
            pltpu.VMEM((16,), jnp.int32),
            pltpu.VMEM((16,), jnp.int32),
            pltpu.VMEM((16,), jnp.int32),
            pltpu.SemaphoreType.DMA,
            pltpu.SemaphoreType.DMA,
            pltpu.SemaphoreType.DMA,
            pltpu.SemaphoreType.DMA,
            pltpu.SemaphoreType.DMA,
            pltpu.SemaphoreType.DMA,
        ],
    )(_sc_body)
    src_t = jnp.transpose(elmo_src, (1, 0, 2))
    tgt_t = jnp.transpose(elmo_tgt, (1, 0, 2))
    outs = kern(src_t, tgt_t)
    return tuple(o.reshape(_B, _L, _D) for o in outs)
